# Initial kernel scaffold; baseline (speedup 1.0000x reference)
import jax
import jax.numpy as jnp
from jax.experimental import pallas as pl


def kernel(logits, target_indices):
    raise NotImplementedError("write your pallas kernel here")


# speedup vs baseline: 21.7759x; 21.7759x over previous
import functools

import jax
import jax.numpy as jnp
import numpy as np
from jax.experimental import pallas as pl
from jax.experimental.pallas import tpu as pltpu

_BINS = 10
_B, _C = 16384, 1000
_RBLK = 512

_EDGES32 = (np.arange(1, _BINS, dtype=np.float32) / np.float32(_BINS)).astype(np.float64)
_LOGIT = np.log(_EDGES32 / (1.0 - _EDGES32)).astype(np.float32)


def _ghm_kernel(tgt_ref, x_ref, out_ref, acc_ref):
    i = pl.program_id(0)

    @pl.when(i == 0)
    def _init():
        for k in range(19):
            acc_ref[k] = jnp.float32(0.0)

    l = x_ref[...]
    col = jax.lax.broadcasted_iota(jnp.int32, l.shape, 1)
    tgt = tgt_ref[...]
    x = jnp.where(col == tgt, -l, l)
    x = jnp.where(col < _C, x, jnp.float32(-1e9))
    loss = jnp.maximum(x, 0.0) + jnp.log1p(jnp.exp(-jnp.abs(x)))

    acc_ref[0] = acc_ref[0] + jnp.sum(loss)
    for k in range(9):
        m = (x >= _LOGIT[k]).astype(jnp.float32)
        acc_ref[1 + k] = acc_ref[1 + k] + jnp.sum(m)
        acc_ref[10 + k] = acc_ref[10 + k] + jnp.sum(m * loss)

    @pl.when(i == pl.num_programs(0) - 1)
    def _fin():
        tot = jnp.float32(_B * _C)
        loss_sum = jnp.float32(0.0)
        n = jnp.float32(0.0)
        for b in range(_BINS):
            cc_lo = tot if b == 0 else acc_ref[b]
            cc_hi = jnp.float32(0.0) if b == 9 else acc_ref[b + 1]
            cs_lo = acc_ref[0] if b == 0 else acc_ref[9 + b]
            cs_hi = jnp.float32(0.0) if b == 9 else acc_ref[10 + b]
            cnt = cc_lo - cc_hi
            sb = cs_lo - cs_hi
            nonempty = cnt > 0.0
            n = n + jnp.where(nonempty, 1.0, 0.0).astype(jnp.float32)
            loss_sum = loss_sum + jnp.where(
                nonempty, sb / jnp.maximum(cnt, 1.0), 0.0
            ).astype(jnp.float32)
        out_ref[0] = loss_sum / jnp.maximum(n, 1.0)


@jax.jit
def kernel(logits, target_indices):
    tgt2d = target_indices.astype(jnp.int32).reshape(_B, 1)
    grid = _B // _RBLK
    out = pl.pallas_call(
        _ghm_kernel,
        grid=(grid,),
        in_specs=[
            pl.BlockSpec((_RBLK, 1), lambda i: (i, 0)),
            pl.BlockSpec((_RBLK, _C), lambda i: (i, 0)),
        ],
        out_specs=pl.BlockSpec(memory_space=pltpu.SMEM),
        out_shape=jax.ShapeDtypeStruct((1,), jnp.float32),
        scratch_shapes=[pltpu.SMEM((19,), jnp.float32)],
    )(tgt2d, logits)
    return out[0]
